# Initial kernel scaffold; baseline (speedup 1.0000x reference)
import jax
import jax.numpy as jnp
from jax.experimental import pallas as pl


def kernel(x, edge_index, W1, b1, Wmu, bmu, Wlv, blv, eps):
    raise NotImplementedError("write your pallas kernel here")


# speedup vs baseline: 8.3139x; 8.3139x over previous
import functools

import jax
import jax.numpy as jnp
from jax import lax
from jax.experimental import pallas as pl
from jax.experimental.pallas import tpu as pltpu
from jax.experimental.pallas import tpu_sc as plsc

N = 10000
E = 160000
D = 256
NPAD = 10240
NB = 80
PAD_E = 163840
RPT = NPAD // 16

f32 = jnp.float32
i32 = jnp.int32

_MESH = plsc.VectorSubcoreMesh(core_axis_name="c", subcore_axis_name="s")


def _zero_vmem_2d(ref, rows, width):
    zero = jnp.zeros((16,), f32)
    groups = width // 16

    def body(i, _):
        for g in range(groups):
            ref[i, pl.ds(g * 16, 16)] = zero
        return 0

    lax.fori_loop(0, rows, body, 0)


def _fill_rows(ref, rows, vec16):
    def body(i, _):
        ref[i, :] = vec16
        return 0

    lax.fori_loop(0, rows, body, 0)


@functools.partial(
    pl.kernel,
    out_type=jax.ShapeDtypeStruct((2, NPAD, 128), f32),
    mesh=_MESH,
    scratch_types=[
        pltpu.VMEM((40, 128), i32),
        pltpu.VMEM((128, 128), f32),
        pltpu.VMEM_SHARED((NPAD, 128), f32),
        pltpu.SemaphoreType.DMA,
    ],
)
def _sc_deg(colg_hbm, out_hbm, colw_v, upd_v, acc, sem):
    del sem
    c = lax.axis_index("c")
    s = lax.axis_index("s")
    w = c * 16 + s
    _zero_vmem_2d(upd_v, 128, 128)
    for k in range(RPT // 128):
        pltpu.sync_copy(upd_v, acc.at[pl.ds(s * RPT + k * 128, 128)])
    e0 = jnp.where(lax.iota(i32, 16) == 0, 1.0, 0.0).astype(f32)

    def fill(i, _):
        upd_v[i, pl.ds(0, 16)] = e0
        return 0

    lax.fori_loop(0, 128, fill, 0)
    pltpu.sync_copy(colg_hbm.at[w // 2, pl.ds((w % 2) * 40, 40)], colw_v)
    plsc.subcore_barrier()

    def body(j, _):
        pltpu.sync_copy(upd_v, acc.at[colw_v.at[j]], add=True)
        return 0

    lax.fori_loop(0, 40, body, 0)
    plsc.subcore_barrier()
    for k in range(RPT // 128):
        base = s * RPT + k * 128
        pltpu.sync_copy(acc.at[pl.ds(base, 128)], upd_v)
        pltpu.sync_copy(upd_v, out_hbm.at[c, pl.ds(base, 128)])


@functools.partial(
    pl.kernel,
    out_type=[
        jax.ShapeDtypeStruct((2, NPAD, 128), f32),
        jax.ShapeDtypeStruct((2, NPAD, 128), f32),
    ],
    mesh=_MESH,
    scratch_types=[
        pltpu.VMEM((80, 128), i32),
        pltpu.VMEM((80, 128), i32),
        pltpu.VMEM((128, 128), f32),
        pltpu.VMEM((40, 128), i32),
        pltpu.VMEM((40, 128), i32),
        pltpu.VMEM_SHARED((NPAD, 128), f32),
        pltpu.SemaphoreType.DMA,
    ],
)
def _sc_pass1(rowg_hbm, colg_hbm, dinvt_hbm, g1_hbm,
              s1_hbm, snum_hbm,
              rowg_v, colg_v, gbuf_v, colw_v, roww_v, acc128, sem):
    c = lax.axis_index("c")
    s = lax.axis_index("s")
    w = c * 16 + s
    pltpu.sync_copy(rowg_hbm.at[w], rowg_v)
    pltpu.sync_copy(colg_hbm.at[s], colg_v)
    pltpu.sync_copy(colg_hbm.at[w // 2, pl.ds((w % 2) * 40, 40)], colw_v)
    pltpu.sync_copy(rowg_hbm.at[w // 2, pl.ds((w % 2) * 40, 40)], roww_v)

    _zero_vmem_2d(gbuf_v, 128, 128)
    for k in range(RPT // 128):
        pltpu.sync_copy(gbuf_v, acc128.at[pl.ds(s * RPT + k * 128, 128)])
    plsc.subcore_barrier()

    def row_body(j, _):
        pltpu.async_copy(g1_hbm.at[rowg_v.at[j]], gbuf_v, sem).wait()
        pltpu.sync_copy(gbuf_v, acc128.at[colg_v.at[j]], add=True)
        return 0

    lax.fori_loop(0, 80, row_body, 0)
    plsc.subcore_barrier()
    for k in range(RPT // 128):
        base = s * RPT + k * 128
        pltpu.sync_copy(acc128.at[pl.ds(base, 128)], gbuf_v)
        pltpu.sync_copy(gbuf_v, s1_hbm.at[c, pl.ds(base, 128)])
    plsc.subcore_barrier()

    _zero_vmem_2d(gbuf_v, 128, 128)
    for k in range(RPT // 128):
        pltpu.sync_copy(gbuf_v, acc128.at[pl.ds(s * RPT + k * 128, 128)])
    plsc.subcore_barrier()

    def sca_body(j, _):
        pltpu.async_copy(dinvt_hbm.at[roww_v.at[j]], gbuf_v, sem).wait()
        pltpu.sync_copy(gbuf_v, acc128.at[colw_v.at[j]], add=True)
        return 0

    lax.fori_loop(0, 40, sca_body, 0)
    plsc.subcore_barrier()
    for k in range(RPT // 128):
        base = s * RPT + k * 128
        pltpu.sync_copy(acc128.at[pl.ds(base, 128)], gbuf_v)
        pltpu.sync_copy(gbuf_v, snum_hbm.at[c, pl.ds(base, 128)])


@functools.partial(
    pl.kernel,
    out_type=[
        jax.ShapeDtypeStruct((2, NPAD, 128), f32),
        jax.ShapeDtypeStruct((2, NPAD, 128), f32),
    ],
    mesh=_MESH,
    scratch_types=[
        pltpu.VMEM((80, 128), i32),
        pltpu.VMEM((80, 128), i32),
        pltpu.VMEM((128, 128), f32),
        pltpu.VMEM_SHARED((NPAD, 128), f32),
        pltpu.SemaphoreType.DMA,
    ],
)
def _sc_pass23(rowg_hbm, colg_hbm, g2_hbm, g3_hbm, s2_hbm, s3_hbm,
               rowg_v, colg_v, gbuf_v, acc128, sem):
    c = lax.axis_index("c")
    s = lax.axis_index("s")
    w = c * 16 + s
    pltpu.sync_copy(rowg_hbm.at[w], rowg_v)
    pltpu.sync_copy(colg_hbm.at[s], colg_v)
    for g_hbm, out_hbm in ((g2_hbm, s2_hbm), (g3_hbm, s3_hbm)):
        _zero_vmem_2d(gbuf_v, 128, 128)
        for k in range(RPT // 128):
            pltpu.sync_copy(gbuf_v, acc128.at[pl.ds(s * RPT + k * 128, 128)])
        plsc.subcore_barrier()

        def row_body(j, _, g_hbm=g_hbm):
            pltpu.async_copy(g_hbm.at[rowg_v.at[j]], gbuf_v, sem).wait()
            pltpu.sync_copy(gbuf_v, acc128.at[colg_v.at[j]], add=True)
            return 0

        lax.fori_loop(0, 80, row_body, 0)
        plsc.subcore_barrier()
        for k in range(RPT // 128):
            base = s * RPT + k * 128
            pltpu.sync_copy(acc128.at[pl.ds(base, 128)], gbuf_v)
            pltpu.sync_copy(gbuf_v, out_hbm.at[c, pl.ds(base, 128)])
        plsc.subcore_barrier()


def _tc1_body(deg_ref, x_ref, w1_ref, g1_ref, dinv_ref, dinv16_ref):
    deg = deg_ref[0, :, 0] + deg_ref[1, :, 0] + 1.0
    dinv = lax.rsqrt(jnp.maximum(deg, 1e-12))
    h = jnp.dot(x_ref[...], w1_ref[...], preferred_element_type=f32)
    h = h * dinv[:, None]
    g1_ref[0, :, :] = h[:, 0:128]
    g1_ref[1, :, :] = h[:, 128:256]
    dinv_ref[0, 0, :] = dinv
    lane = lax.broadcasted_iota(i32, (128, 128), 1)
    dinv16_ref[...] = jnp.where(lane == 0, dinv[:, None], 0.0)


def _tc1(deg2, x, W1):
    return pl.pallas_call(
        _tc1_body,
        grid=(NB,),
        compiler_params=pltpu.CompilerParams(vmem_limit_bytes=32 * 1024 * 1024),
        in_specs=[
            pl.BlockSpec((2, 128, 128), lambda rb: (0, rb, 0)),
            pl.BlockSpec((128, D), lambda rb: (rb, 0)),
            pl.BlockSpec((D, D), lambda rb: (0, 0)),
        ],
        out_specs=[
            pl.BlockSpec((2, 128, 128), lambda rb: (0, rb, 0)),
            pl.BlockSpec((1, 1, 128), lambda rb: (rb, 0, 0)),
            pl.BlockSpec((128, 128), lambda rb: (rb, 0)),
        ],
        out_shape=[
            jax.ShapeDtypeStruct((2, NPAD, 128), f32),
            jax.ShapeDtypeStruct((NB, 1, 128), f32),
            jax.ShapeDtypeStruct((NPAD, 128), f32),
        ],
    )(deg2, x, W1)


def _tc2_body(s1_ref, g1_ref, dinv_ref, b1_ref, wmu_ref, wlv_ref,
              g2_ref, g3_ref):
    dinv = dinv_ref[0, 0, :]
    pre = jnp.concatenate(
        [s1_ref[0, :, :] + g1_ref[0, :, :], s1_ref[1, :, :] + g1_ref[1, :, :]],
        axis=1)
    hidden = jnp.maximum(pre * dinv[:, None] + b1_ref[0, :], 0.0)
    g2 = jnp.dot(hidden, wmu_ref[...], preferred_element_type=f32) * dinv[:, None]
    g3 = jnp.dot(hidden, wlv_ref[...], preferred_element_type=f32) * dinv[:, None]
    g2_ref[0, :, :] = g2[:, 0:128]
    g2_ref[1, :, :] = g2[:, 128:256]
    g3_ref[0, :, :] = g3[:, 0:128]
    g3_ref[1, :, :] = g3[:, 128:256]


def _tc2(S1, G1, dinv3d, b1, Wmu, Wlv):
    return pl.pallas_call(
        _tc2_body,
        grid=(NB,),
        compiler_params=pltpu.CompilerParams(vmem_limit_bytes=32 * 1024 * 1024),
        in_specs=[
            pl.BlockSpec((2, 128, 128), lambda rb: (0, rb, 0)),
            pl.BlockSpec((2, 128, 128), lambda rb: (0, rb, 0)),
            pl.BlockSpec((1, 1, 128), lambda rb: (rb, 0, 0)),
            pl.BlockSpec((1, D), lambda rb: (0, 0)),
            pl.BlockSpec((D, D), lambda rb: (0, 0)),
            pl.BlockSpec((D, D), lambda rb: (0, 0)),
        ],
        out_specs=[
            pl.BlockSpec((2, 128, 128), lambda rb: (0, rb, 0)),
            pl.BlockSpec((2, 128, 128), lambda rb: (0, rb, 0)),
        ],
        out_shape=[
            jax.ShapeDtypeStruct((2, NPAD, 128), f32),
            jax.ShapeDtypeStruct((2, NPAD, 128), f32),
        ],
    )(S1, G1, dinv3d, b1, Wmu, Wlv)


def _tc3_body(s2_ref, g2_ref, s3_ref, g3_ref, dinv_ref, snum_ref, x_ref,
              b1_ref, bmu_ref, blv_ref, wmu_ref,
              mu_ref, lv_ref, rec_ref, loss_ref):
    rb = pl.program_id(0)
    dinv = dinv_ref[0, 0, :]
    mu = jnp.concatenate(
        [s2_ref[0, :, :] + g2_ref[0, :, :], s2_ref[1, :, :] + g2_ref[1, :, :]],
        axis=1) * dinv[:, None] + bmu_ref[0, :]
    logvar = jnp.concatenate(
        [s3_ref[0, :, :] + g3_ref[0, :, :], s3_ref[1, :, :] + g3_ref[1, :, :]],
        axis=1) * dinv[:, None] + blv_ref[0, :]
    mu_ref[...] = mu
    lv_ref[...] = logvar

    snum = snum_ref[0, :, 0] + snum_ref[1, :, 0]
    sfac = dinv * (snum + dinv)
    v = jnp.sum(jnp.maximum(b1_ref[...], 0.0).reshape(D, 1) * wmu_ref[...],
                axis=0)
    recon = jax.nn.sigmoid(sfac[:, None] * v[None, :] + bmu_ref[0, :])
    rec_ref[...] = recon

    rowid = rb * 128 + lax.broadcasted_iota(i32, (128, D), 0)
    valid = rowid < N
    rc = jnp.clip(recon, 1e-7, 1.0 - 1e-7)
    xb = x_ref[...]
    bce_terms = -(xb * jnp.log(rc) + (1.0 - xb) * jnp.log(1.0 - rc))
    bce_part = jnp.sum(jnp.where(valid, bce_terms, 0.0))
    kl_terms = 1.0 + logvar - mu * mu - jnp.exp(logvar)
    kl_part = jnp.sum(jnp.where(valid, kl_terms, 0.0))
    part = bce_part / (N * D) - 0.5 * kl_part

    @pl.when(rb == 0)
    def _():
        loss_ref[:, :] = jnp.zeros((1, 1), f32)

    loss_ref[:, :] = loss_ref[:, :] + part


def _tc3(S2, G2, S3, G3, dinv3d, snum, x, b1, bmu, blv, Wmu):
    return pl.pallas_call(
        _tc3_body,
        grid=(NB,),
        compiler_params=pltpu.CompilerParams(vmem_limit_bytes=32 * 1024 * 1024),
        in_specs=[
            pl.BlockSpec((2, 128, 128), lambda rb: (0, rb, 0)),
            pl.BlockSpec((2, 128, 128), lambda rb: (0, rb, 0)),
            pl.BlockSpec((2, 128, 128), lambda rb: (0, rb, 0)),
            pl.BlockSpec((2, 128, 128), lambda rb: (0, rb, 0)),
            pl.BlockSpec((1, 1, 128), lambda rb: (rb, 0, 0)),
            pl.BlockSpec((2, 128, 128), lambda rb: (0, rb, 0)),
            pl.BlockSpec((128, D), lambda rb: (rb, 0)),
            pl.BlockSpec((1, D), lambda rb: (0, 0)),
            pl.BlockSpec((1, D), lambda rb: (0, 0)),
            pl.BlockSpec((1, D), lambda rb: (0, 0)),
            pl.BlockSpec((D, D), lambda rb: (0, 0)),
        ],
        out_specs=[
            pl.BlockSpec((128, D), lambda rb: (rb, 0)),
            pl.BlockSpec((128, D), lambda rb: (rb, 0)),
            pl.BlockSpec((128, D), lambda rb: (rb, 0)),
            pl.BlockSpec((1, 1), lambda rb: (0, 0)),
        ],
        out_shape=[
            jax.ShapeDtypeStruct((NPAD, D), f32),
            jax.ShapeDtypeStruct((NPAD, D), f32),
            jax.ShapeDtypeStruct((NPAD, D), f32),
            jax.ShapeDtypeStruct((1, 1), f32),
        ],
    )(S2, G2, S3, G3, dinv3d, snum, x, b1, bmu, blv, Wmu)


def kernel(x, edge_index, W1, b1, Wmu, bmu, Wlv, blv, eps):
    del eps
    row = edge_index[0].astype(i32)
    col = edge_index[1].astype(i32)
    pad = PAD_E - E
    rowp = jnp.concatenate([row, jnp.zeros((pad,), i32)])
    colp = jnp.concatenate([col, jnp.full((pad,), NPAD - 1, i32)])

    rg = rowp.reshape(16, 80, 128)
    rowg = jnp.concatenate([rg[None], rg[None] + NPAD], axis=0).reshape(32, 80, 128)
    colg = colp.reshape(16, 80, 128)

    deg2 = _sc_deg(colg)

    G1, dinv3d, dinv16 = _tc1(deg2, x, W1)

    S1, snum = _sc_pass1(rowg, colg, dinv16,
                         G1.reshape(2 * NPAD, 128))

    b1r = b1.reshape(1, D)
    G2, G3 = _tc2(S1, G1, dinv3d, b1r, Wmu, Wlv)

    S2, S3 = _sc_pass23(rowg, colg, G2.reshape(2 * NPAD, 128),
                        G3.reshape(2 * NPAD, 128))

    mu, logvar, recon, loss = _tc3(S2, G2, S3, G3, dinv3d, snum, x,
                                   b1r, bmu.reshape(1, D), blv.reshape(1, D),
                                   Wmu)
    return recon[:N], mu[:N], logvar[:N], loss[0, 0]
